# baseline (speedup 1.0000x reference)
import functools

import jax
import jax.numpy as jnp
from jax import lax
from jax.experimental import pallas as pl
from jax.experimental.pallas import tpu as pltpu
from jax.experimental.pallas import tpu_sc as plsc

_TOP = 1000
_PAD = 1024
_DUMP = 1040
_DSTRIDE = 1056
_NEG = -1e9


def _monotone_i32(x):
    b = lax.bitcast_convert_type(x, jnp.int32)
    return b ^ ((b >> 31) & jnp.int32(0x7FFFFFFF))


def _select_body(sup_ref, logits_ref, lse_ref, slot_ref):
    S, V = logits_ref.shape[1], logits_ref.shape[2]
    x = logits_ref[0]
    col = lax.broadcasted_iota(jnp.int32, (S, V), 1)
    supmask = jnp.zeros((S, V), dtype=jnp.bool_)
    for i in range(sup_ref.shape[0]):
        supmask = supmask | (col == sup_ref[i])
    x = jnp.where(supmask, _NEG, x)
    m = _monotone_i32(x)

    def cnt(t):
        return jnp.sum((m >= t).astype(jnp.int32), axis=1, keepdims=True)

    imin = jnp.int32(-2147483648)
    imax = jnp.int32(2147483647)
    zero = jnp.zeros((S, 1), jnp.int32)
    c0 = cnt(zero)
    negside = c0 < _TOP
    lo = jnp.where(negside, imin, zero)
    hi = jnp.where(negside, jnp.int32(-1), imax)
    clo = jnp.where(negside, jnp.int32(V), c0)
    chi = jnp.where(negside, c0, jnp.int32(0))

    def cond(carry):
        lo, hi, clo, chi, it = carry
        return jnp.any(lo < hi)

    def body(carry):
        lo, hi, clo, chi, it = carry
        width = hi - lo
        mid = lo + ((width >> 1) + (width & 1))
        frac = (clo.astype(jnp.float32) - _TOP) / (
            clo.astype(jnp.float32) - chi.astype(jnp.float32))
        frac = jnp.clip(frac, 0.0, 0.999)
        sec = lo + 1 + (frac * (width - 1).astype(jnp.float32)).astype(jnp.int32)
        sec = jnp.clip(sec, lo + 1, hi)
        p = jnp.where((it & 1) == 0, sec, mid)
        act = lo < hi
        c = cnt(p)
        ok = act & (c >= _TOP)
        nok = act & jnp.logical_not(c >= _TOP)
        lo = jnp.where(ok, p, lo)
        clo = jnp.where(ok, c, clo)
        hi = jnp.where(nok, p - 1, hi)
        chi = jnp.where(nok, c, chi)
        return lo, hi, clo, chi, it + 1

    lo, hi, clo, chi, _ = lax.while_loop(
        cond, body, (lo, hi, clo, chi, jnp.int32(0)))
    tstar = lo

    G = jnp.sum((m > tstar).astype(jnp.int32), axis=1, keepdims=True)
    E = _TOP - G
    eqmask = m == tstar
    eqcnt = jnp.sum(eqmask.astype(jnp.int32), axis=1, keepdims=True)

    triv = eqcnt == E
    lo2 = jnp.where(triv, jnp.int32(V - 1), jnp.int32(0))
    hi2 = jnp.full((S, 1), V - 1, jnp.int32)

    def cond2(carry):
        lo2, hi2 = carry
        return jnp.any(lo2 < hi2)

    def body2(carry):
        lo2, hi2 = carry
        mid = (lo2 + hi2) >> 1
        ec = jnp.sum((eqmask & (col <= mid)).astype(jnp.int32),
                     axis=1, keepdims=True)
        act = lo2 < hi2
        ok = act & (ec >= E)
        hi2 = jnp.where(ok, mid, hi2)
        lo2 = jnp.where(act & jnp.logical_not(ec >= E), mid + 1, lo2)
        return lo2, hi2

    lo2, _ = lax.while_loop(cond2, body2, (lo2, hi2))
    cstar = lo2

    sel = (m > tstar) | (eqmask & (col <= cstar))
    M = jnp.max(x, axis=1, keepdims=True)
    sumexp = jnp.sum(jnp.where(sel, jnp.exp(x - M), 0.0), axis=1, keepdims=True)
    lse = jnp.log(sumexp) + M

    tbits = jnp.where(tstar >= 0, tstar, tstar ^ jnp.int32(0x7FFFFFFF))
    tf = lax.bitcast_convert_type(tbits, jnp.float32)

    lse_ref[...] = jnp.broadcast_to(lse, (S, 128))[None]
    selI = jnp.where(sel, 1, 0)
    acc = selI
    d = 1
    while d < V:
        sh = jnp.where(col >= d, pltpu.roll(acc, d, 1), 0)
        acc = acc + sh
        d <<= 1
    prefix = acc - selI
    b = pl.program_id(0)
    rowbase = (b * S + lax.broadcasted_iota(jnp.int32, (S, 1), 0)) * _DSTRIDE
    slot_ref[...] = (rowbase + jnp.where(sel, prefix, _DUMP))[None]


def _select(logits, token_ids_to_suppress):
    B, S, V = logits.shape
    return pl.pallas_call(
        _select_body,
        grid_spec=pltpu.PrefetchScalarGridSpec(
            num_scalar_prefetch=1,
            grid=(B,),
            in_specs=[pl.BlockSpec((1, S, V), lambda b, sup: (b, 0, 0))],
            out_specs=[
                pl.BlockSpec((1, S, 128), lambda b, sup: (b, 0, 0)),
                pl.BlockSpec((1, S, V), lambda b, sup: (b, 0, 0)),
            ],
        ),
        out_shape=[
            jax.ShapeDtypeStruct((B, S, 128), jnp.float32),
            jax.ShapeDtypeStruct((B, S, V), jnp.int32),
        ],
    )(token_ids_to_suppress, logits)


def _extract(logits2d, slots2d, iota2d):
    BS, V = logits2d.shape
    mesh = plsc.VectorSubcoreMesh(core_axis_name="c", subcore_axis_name="s")
    info = plsc.get_sparse_core_info()
    NC, NS, L = info.num_cores, info.num_subcores, info.num_lanes
    NW = NC * NS
    rows_per_w = BS // NW
    NCH = V // 128
    BURST = 8

    @functools.partial(
        pl.kernel,
        mesh=mesh,
        out_type=[
            jax.ShapeDtypeStruct((BS * _DSTRIDE,), jnp.float32),
            jax.ShapeDtypeStruct((BS * _DSTRIDE,), jnp.int32),
        ],
        scratch_types=[
            pltpu.VMEM((NCH, 128), jnp.float32),
            pltpu.VMEM((NCH, 128), jnp.int32),
            pltpu.VMEM((NCH, 128), jnp.int32),
            pltpu.SemaphoreType.DMA,
        ],
    )
    def sc_kernel(logits_hbm, slots_hbm, iota_hbm, ovals_hbm, oidx_hbm,
                  row_v, srow_v, iota_v, sem):
        wid = lax.axis_index("s") * NC + lax.axis_index("c")
        pltpu.sync_copy(iota_hbm.at[0], iota_v)

        def row_body(r, _):
            row = wid * rows_per_w + r
            pltpu.sync_copy(logits_hbm.at[row], row_v)
            pltpu.sync_copy(slots_hbm.at[row], srow_v)

            def burst(bi, _):
                copies = []
                for u in range(BURST):
                    j = bi * BURST + u
                    copies.append(pltpu.async_copy(
                        row_v.at[j], ovals_hbm.at[srow_v.at[j]], sem))
                    copies.append(pltpu.async_copy(
                        iota_v.at[j], oidx_hbm.at[srow_v.at[j]], sem))
                for c in copies:
                    c.wait()
                return 0

            lax.fori_loop(0, NCH // BURST, burst, 0)
            tail = []
            for j in range((NCH // BURST) * BURST, NCH):
                tail.append(pltpu.async_copy(
                    row_v.at[j], ovals_hbm.at[srow_v.at[j]], sem))
                tail.append(pltpu.async_copy(
                    iota_v.at[j], oidx_hbm.at[srow_v.at[j]], sem))
            for c in tail:
                c.wait()
            return 0

        lax.fori_loop(0, rows_per_w, row_body, 0)

    return sc_kernel(logits2d.reshape(BS, NCH, 128),
                     slots2d.reshape(BS, NCH, 128), iota2d)


def _lex_gt(av, ai, bv, bi):
    return (av > bv) | ((av == bv) & (ai < bi))


def _finish_body(cv_ref, ci_ref, g_ref, lse_ref, pos_ref, x_ref,
                 am_ref, lp_ref, tok_ref, fx_ref, fam_ref, fpos_ref):
    S = cv_ref.shape[1]
    N = cv_ref.shape[2]
    v = cv_ref[0]
    ii = ci_ref[0]
    lanes = lax.broadcasted_iota(jnp.int32, (S, N), 1)
    pre = lanes < _TOP
    v = jnp.where(pre, v, jnp.float32(-3e38))
    ii = jnp.where(pre, ii, jnp.int32(2147483647))

    k = 2
    while k <= N:
        j = k >> 1
        while j >= 1:
            pv = jnp.where((lanes & j) != 0,
                           pltpu.roll(v, j, 1), pltpu.roll(v, N - j, 1))
            pi = jnp.where((lanes & j) != 0,
                           pltpu.roll(ii, j, 1), pltpu.roll(ii, N - j, 1))
            i_high = (lanes & j) != 0
            asc = (lanes & k) != 0
            pg = _lex_gt(pv, pi, v, ii)
            take = pg ^ i_high ^ asc
            v = jnp.where(take, pv, v)
            ii = jnp.where(take, pi, ii)
            j >>= 1
        k <<= 1

    lse = lse_ref[0][:, :1]
    lp = v - lse
    y = jnp.where(lanes < _TOP, lp + g_ref[0], jnp.float32(-3e38))
    rowmax = jnp.max(y, axis=1, keepdims=True)
    winpos = jnp.min(jnp.where(y == rowmax, lanes, jnp.int32(N)),
                     axis=1, keepdims=True)
    onewin = lanes == winpos
    chosen_lp = jnp.sum(jnp.where(onewin, lp, 0.0), axis=1, keepdims=True)
    tok = jnp.sum(jnp.where(onewin, ii, 0), axis=1, keepdims=True)
    lp_ref[...] = jnp.broadcast_to(chosen_lp, (S, 128))[None]
    tok_ref[...] = jnp.broadcast_to(tok, (S, 128))[None]

    rep = lambda a: jnp.concatenate([a] * (128 // S), axis=1)
    p = rep(pos_ref[0])
    xs = rep(x_ref[0])
    am = rep(am_ref[0])
    l16 = lax.broadcasted_iota(jnp.int32, (1, 128), 1) % S
    rank = jnp.zeros((1, 128), jnp.int32)
    for sh in range(1, S):
        q = pltpu.roll(p, sh, 1)
        before = l16 >= sh
        rank = rank + ((q < p) | ((q == p) & before)).astype(jnp.int32)
    fp = jnp.zeros((1, 128), jnp.int32)
    fxv = jnp.zeros((1, 128), jnp.int32)
    fam = jnp.zeros((1, 128), jnp.int32)
    for sh in range(S):
        rr = pltpu.roll(rank, sh, 1)
        hit = rr == l16
        fp = fp + jnp.where(hit, pltpu.roll(p, sh, 1), 0)
        fxv = fxv + jnp.where(hit, pltpu.roll(xs, sh, 1), 0)
        fam = fam + jnp.where(hit, pltpu.roll(am, sh, 1), 0)
    fpos_ref[...] = fp[:, :S][None]
    fx_ref[...] = fxv[:, :S][None]
    fam_ref[...] = fam[:, :S][None]


def _finish(cvals, cidx, g, lse, positions, x, attention_mask):
    B = positions.shape[0]
    S = positions.shape[1]
    cvals = cvals.reshape(B, S, _PAD)
    cidx = cidx.reshape(B, S, _PAD)
    pos3 = positions.reshape(B, 1, S)
    x3 = x.reshape(B, 1, S)
    am3 = attention_mask.astype(jnp.int32).reshape(B, 1, S)
    bs = lambda b: (b, 0, 0)
    return pl.pallas_call(
        _finish_body,
        grid_spec=pltpu.PrefetchScalarGridSpec(
            num_scalar_prefetch=0,
            grid=(B,),
            in_specs=[
                pl.BlockSpec((1, S, _PAD), bs),
                pl.BlockSpec((1, S, _PAD), bs),
                pl.BlockSpec((1, S, _PAD), bs),
                pl.BlockSpec((1, S, 128), bs),
                pl.BlockSpec((1, 1, S), bs),
                pl.BlockSpec((1, 1, S), bs),
                pl.BlockSpec((1, 1, S), bs),
            ],
            out_specs=[
                pl.BlockSpec((1, S, 128), bs),
                pl.BlockSpec((1, S, 128), bs),
                pl.BlockSpec((1, 1, S), bs),
                pl.BlockSpec((1, 1, S), bs),
                pl.BlockSpec((1, 1, S), bs),
            ],
        ),
        out_shape=[
            jax.ShapeDtypeStruct((B, S, 128), jnp.float32),
            jax.ShapeDtypeStruct((B, S, 128), jnp.int32),
            jax.ShapeDtypeStruct((B, 1, S), jnp.int32),
            jax.ShapeDtypeStruct((B, 1, S), jnp.int32),
            jax.ShapeDtypeStruct((B, 1, S), jnp.int32),
        ],
    )(cvals, cidx, g, lse, pos3, x3, am3)


def kernel(logits, x, positions, attention_mask, token_ids_to_suppress, top):
    B, S, V = logits.shape

    skey = jax.random.fold_in(jax.random.key(0), 1)
    g = jax.random.gumbel(skey, (B, S, _TOP), jnp.float32)
    g = jnp.pad(g, ((0, 0), (0, 0), (0, _PAD - _TOP)),
                constant_values=-3e38)

    lse, slots = _select(logits, token_ids_to_suppress)
    iota2d = jnp.arange(V, dtype=jnp.int32).reshape(1, V // 128, 128)
    cvals, cidx = _extract(logits.reshape(B * S, V), slots.reshape(B * S, V),
                           iota2d)
    cvals = cvals.reshape(B * S, _DSTRIDE)[:, :_PAD]
    cidx = cidx.reshape(B * S, _DSTRIDE)[:, :_PAD]
    lp, tok, fx, fam, fpos = _finish(
        cvals, cidx, g, lse, positions, x, attention_mask)

    chosen_logprob = lp[:, :, 0]
    sampled_tokens = tok[:, :, 0]
    final_x = fx.reshape(B, S)
    final_attention_mask = fam.reshape(B, S).astype(jnp.bool_)
    final_positions = fpos.reshape(B, S)
    return (chosen_logprob, sampled_tokens, final_x,
            final_attention_mask, final_positions)
